# baseline (speedup 1.0000x reference)
import functools

import jax
import jax.numpy as jnp
from jax.experimental import pallas as pl

_NEG_SLOPE = 0.2


def _augment(g, f, n_heads):
    bm = g.shape[0]
    parts = []
    for h in range(n_heads):
        parts.append(g[:, h * f:(h + 1) * f].astype(jnp.bfloat16))
        parts.append(jnp.ones((bm, 1), jnp.bfloat16))
        parts.append(jnp.zeros((bm, 128 - f - 1), jnp.bfloat16))
    return jnp.concatenate(parts, axis=1)


def _proj_body(x_ref, w_ref, a_ref, gext_ref, el_ref, er_ref, *, n_heads, f1):
    g = jnp.dot(x_ref[...], w_ref[...], preferred_element_type=jnp.float32)
    for h in range(n_heads):
        g_h = g[:, h * f1:(h + 1) * f1]
        el_ref[h, :] = jnp.sum(g_h * a_ref[h, :f1][None, :], axis=1)
        er_ref[h, :] = jnp.sum(g_h * a_ref[h, f1:][None, :], axis=1)
    gext_ref[...] = _augment(g, f1, n_heads)


def _attend(el, er, adj, gext_h, f):
    mel = jnp.max(el)
    mer = jnp.max(er)
    e1l = jnp.exp(el - mel).astype(jnp.bfloat16)
    e2l = jnp.exp(_NEG_SLOPE * el - mel).astype(jnp.bfloat16)
    e1r = jnp.exp(er - mer).astype(jnp.bfloat16)
    e2r = jnp.exp(_NEG_SLOPE * er - mer).astype(jnp.bfloat16)
    t = jnp.maximum(e1l[:, None] * e1r[None, :], e2l[:, None] * e2r[None, :])
    p = jnp.where(adj, t, jnp.bfloat16(0.0))
    oe = jnp.dot(p, gext_h, preferred_element_type=jnp.float32)
    return oe[:, :f] / oe[:, f:f + 1]


def _layer1_body(el_ref, er_ref, adj_ref, gext_ref, w2_ref, g2_ref, g2ext_ref,
                 *, n_heads, f1):
    adj = adj_ref[...] != 0
    acc = None
    for h in range(n_heads):
        o = _attend(el_ref[h, :], er_ref[h, :], adj,
                    gext_ref[:, h * 128:(h + 1) * 128], f1)
        acc = o if acc is None else acc + o
    hmean = acc * (1.0 / n_heads)
    he = jnp.where(hmean > 0.0, hmean, jnp.exp(hmean) - 1.0)
    g2 = jnp.dot(he, w2_ref[...], preferred_element_type=jnp.float32)
    g2_ref[...] = g2
    g2ext_ref[...] = _augment(g2, g2.shape[1], 1)


def _layer2_body(g2_blk_ref, g2_all_ref, adj_ref, g2ext_ref, a2_ref, out_ref,
                 *, c):
    a_l = a2_ref[0, :c]
    a_r = a2_ref[0, c:]
    el = jnp.sum(g2_blk_ref[...] * a_l[None, :], axis=1)
    er = jnp.sum(g2_all_ref[...] * a_r[None, :], axis=1)
    out_ref[...] = _attend(el, er, adj_ref[...] != 0, g2ext_ref[...], c)


def kernel(x, adj_mat, W1, a1, W2, a2):
    n, fin = x.shape
    htot = W1.shape[1]
    n_heads = a1.shape[0]
    f1 = htot // n_heads
    c = W2.shape[1]
    adj = adj_mat.view(jnp.int8).reshape(n, n)
    bm = 2048 if n % 2048 == 0 else n
    grid = (n // bm,)

    gext1, el1, er1 = pl.pallas_call(
        functools.partial(_proj_body, n_heads=n_heads, f1=f1),
        grid=grid,
        in_specs=[
            pl.BlockSpec((bm, fin), lambda i: (i, 0)),
            pl.BlockSpec((fin, htot), lambda i: (0, 0)),
            pl.BlockSpec((n_heads, 2 * f1), lambda i: (0, 0)),
        ],
        out_specs=[
            pl.BlockSpec((bm, n_heads * 128), lambda i: (i, 0)),
            pl.BlockSpec((n_heads, bm), lambda i: (0, i)),
            pl.BlockSpec((n_heads, bm), lambda i: (0, i)),
        ],
        out_shape=[
            jax.ShapeDtypeStruct((n, n_heads * 128), jnp.bfloat16),
            jax.ShapeDtypeStruct((n_heads, n), jnp.float32),
            jax.ShapeDtypeStruct((n_heads, n), jnp.float32),
        ],
    )(x.astype(jnp.bfloat16), W1.astype(jnp.bfloat16), a1)

    g2, g2ext = pl.pallas_call(
        functools.partial(_layer1_body, n_heads=n_heads, f1=f1),
        grid=grid,
        in_specs=[
            pl.BlockSpec((n_heads, bm), lambda i: (0, i)),
            pl.BlockSpec((n_heads, n), lambda i: (0, 0)),
            pl.BlockSpec((bm, n), lambda i: (i, 0)),
            pl.BlockSpec((n, n_heads * 128), lambda i: (0, 0)),
            pl.BlockSpec((f1, c), lambda i: (0, 0)),
        ],
        out_specs=[
            pl.BlockSpec((bm, c), lambda i: (i, 0)),
            pl.BlockSpec((bm, 128), lambda i: (i, 0)),
        ],
        out_shape=[
            jax.ShapeDtypeStruct((n, c), jnp.float32),
            jax.ShapeDtypeStruct((n, 128), jnp.bfloat16),
        ],
    )(el1, er1, adj, gext1, W2)

    out = pl.pallas_call(
        functools.partial(_layer2_body, c=c),
        grid=grid,
        in_specs=[
            pl.BlockSpec((bm, c), lambda i: (i, 0)),
            pl.BlockSpec((n, c), lambda i: (0, 0)),
            pl.BlockSpec((bm, n), lambda i: (i, 0)),
            pl.BlockSpec((n, 128), lambda i: (0, 0)),
            pl.BlockSpec((1, 2 * c), lambda i: (0, 0)),
        ],
        out_specs=pl.BlockSpec((bm, c), lambda i: (i, 0)),
        out_shape=jax.ShapeDtypeStruct((n, c), jnp.float32),
    )(g2, g2, adj, g2ext, a2)

    return out


# speedup vs baseline: 1.3570x; 1.3570x over previous
import functools

import jax
import jax.numpy as jnp
from jax.experimental import pallas as pl

_NEG_SLOPE = 0.2


def _augment(g, f, n_heads):
    bm = g.shape[0]
    parts = []
    for h in range(n_heads):
        parts.append(g[:, h * f:(h + 1) * f].astype(jnp.bfloat16))
        parts.append(jnp.ones((bm, 1), jnp.bfloat16))
        parts.append(jnp.zeros((bm, 128 - f - 1), jnp.bfloat16))
    return jnp.concatenate(parts, axis=1)


def _proj_body(x_ref, w_ref, a_ref, gext_ref, el_ref, er_ref, *, n_heads, f1):
    g = jnp.dot(x_ref[...], w_ref[...], preferred_element_type=jnp.float32)
    for h in range(n_heads):
        g_h = g[:, h * f1:(h + 1) * f1]
        el_ref[h, :] = jnp.sum(g_h * a_ref[h, :f1][None, :], axis=1)
        er_ref[h, :] = jnp.sum(g_h * a_ref[h, f1:][None, :], axis=1)
    gext_ref[...] = _augment(g, f1, n_heads)


def _attend(el, er, adj, gext_h, f):
    mel = jnp.max(el)
    mer = jnp.max(er)
    e1l = jnp.exp(el - mel).astype(jnp.bfloat16)
    e2l = jnp.exp(_NEG_SLOPE * el - mel).astype(jnp.bfloat16)
    e1r = jnp.exp(er - mer).astype(jnp.bfloat16)
    e2r = jnp.exp(_NEG_SLOPE * er - mer).astype(jnp.bfloat16)
    t = jnp.maximum(e1l[:, None] * e1r[None, :], e2l[:, None] * e2r[None, :])
    p = jnp.where(adj, t, jnp.bfloat16(0.0))
    oe = jnp.dot(p, gext_h, preferred_element_type=jnp.float32)
    return oe[:, :f] / oe[:, f:f + 1]


def _layer1_body(el_ref, er_ref, adj_ref, gext_ref, w2_ref, g2_ref, g2ext_ref,
                 *, n_heads, f1):
    adj = adj_ref[...] != 0
    acc = None
    for h in range(n_heads):
        o = _attend(el_ref[h, :], er_ref[h, :], adj,
                    gext_ref[:, h * 128:(h + 1) * 128], f1)
        acc = o if acc is None else acc + o
    hmean = acc * (1.0 / n_heads)
    he = jnp.where(hmean > 0.0, hmean, jnp.exp(hmean) - 1.0)
    g2 = jnp.dot(he, w2_ref[...], preferred_element_type=jnp.float32)
    g2_ref[...] = g2
    g2ext_ref[...] = _augment(g2, g2.shape[1], 1)


def _layer2_body(g2_blk_ref, g2_all_ref, adj_ref, g2ext_ref, a2_ref, out_ref,
                 *, c):
    a_l = a2_ref[0, :c]
    a_r = a2_ref[0, c:]
    el = jnp.sum(g2_blk_ref[...] * a_l[None, :], axis=1)
    er = jnp.sum(g2_all_ref[...] * a_r[None, :], axis=1)
    out_ref[...] = _attend(el, er, adj_ref[...] != 0, g2ext_ref[...], c)


def kernel(x, adj_mat, W1, a1, W2, a2):
    n, fin = x.shape
    htot = W1.shape[1]
    n_heads = a1.shape[0]
    f1 = htot // n_heads
    c = W2.shape[1]
    adj = adj_mat.view(jnp.int8).reshape(n, n)
    bm = 1024 if n % 1024 == 0 else n
    grid = (n // bm,)

    gext1, el1, er1 = pl.pallas_call(
        functools.partial(_proj_body, n_heads=n_heads, f1=f1),
        grid=grid,
        in_specs=[
            pl.BlockSpec((bm, fin), lambda i: (i, 0)),
            pl.BlockSpec((fin, htot), lambda i: (0, 0)),
            pl.BlockSpec((n_heads, 2 * f1), lambda i: (0, 0)),
        ],
        out_specs=[
            pl.BlockSpec((bm, n_heads * 128), lambda i: (i, 0)),
            pl.BlockSpec((n_heads, bm), lambda i: (0, i)),
            pl.BlockSpec((n_heads, bm), lambda i: (0, i)),
        ],
        out_shape=[
            jax.ShapeDtypeStruct((n, n_heads * 128), jnp.bfloat16),
            jax.ShapeDtypeStruct((n_heads, n), jnp.float32),
            jax.ShapeDtypeStruct((n_heads, n), jnp.float32),
        ],
    )(x.astype(jnp.bfloat16), W1.astype(jnp.bfloat16), a1)

    g2, g2ext = pl.pallas_call(
        functools.partial(_layer1_body, n_heads=n_heads, f1=f1),
        grid=grid,
        in_specs=[
            pl.BlockSpec((n_heads, bm), lambda i: (0, i)),
            pl.BlockSpec((n_heads, n), lambda i: (0, 0)),
            pl.BlockSpec((bm, n), lambda i: (i, 0)),
            pl.BlockSpec((n, n_heads * 128), lambda i: (0, 0)),
            pl.BlockSpec((f1, c), lambda i: (0, 0)),
        ],
        out_specs=[
            pl.BlockSpec((bm, c), lambda i: (i, 0)),
            pl.BlockSpec((bm, 128), lambda i: (i, 0)),
        ],
        out_shape=[
            jax.ShapeDtypeStruct((n, c), jnp.float32),
            jax.ShapeDtypeStruct((n, 128), jnp.bfloat16),
        ],
    )(el1, er1, adj, gext1, W2)

    out = pl.pallas_call(
        functools.partial(_layer2_body, c=c),
        grid=grid,
        in_specs=[
            pl.BlockSpec((bm, c), lambda i: (i, 0)),
            pl.BlockSpec((n, c), lambda i: (0, 0)),
            pl.BlockSpec((bm, n), lambda i: (i, 0)),
            pl.BlockSpec((n, 128), lambda i: (0, 0)),
            pl.BlockSpec((1, 2 * c), lambda i: (0, 0)),
        ],
        out_specs=pl.BlockSpec((bm, c), lambda i: (i, 0)),
        out_shape=jax.ShapeDtypeStruct((n, c), jnp.float32),
    )(g2, g2, adj, g2ext, a2)

    return out
